# Initial kernel scaffold; baseline (speedup 1.0000x reference)
import jax
import jax.numpy as jnp
from jax.experimental import pallas as pl


def kernel(delta_adj_raw, cost_p_sum, edge_index):
    raise NotImplementedError("write your pallas kernel here")


# speedup vs baseline: 1.0012x; 1.0012x over previous
import jax
import jax.numpy as jnp
from jax.experimental import pallas as pl

N = 100000
E = 6400000
BUDGET = 1000.0
EPS_POS = 1e-10
EPS_NORM = 1e-08


def _w_kernel(x_ref, o_ref):
    x = x_ref[...]
    o_ref[...] = (x * x + EPS_POS) - EPS_POS


def kernel(delta_adj_raw, cost_p_sum, edge_index):
    blk = 512 * 1024
    grid = (E + blk - 1) // blk
    w = pl.pallas_call(
        _w_kernel,
        out_shape=jax.ShapeDtypeStruct((E,), jnp.float32),
        grid=(grid,),
        in_specs=[pl.BlockSpec((blk,), lambda i: (i,))],
        out_specs=pl.BlockSpec((blk,), lambda i: (i,)),
    )(delta_adj_raw)
    s = (w * cost_p_sum).sum()
    scale = BUDGET / jnp.maximum(s, EPS_NORM)
    vals = w * scale
    i = edge_index[0]
    j = edge_index[1]
    ii = jnp.concatenate([i, j], axis=0)
    jj = jnp.concatenate([j, i], axis=0)
    vv = jnp.concatenate([vals, vals], axis=0)
    lin = ii * N + jj
    order = jnp.argsort(lin)
    lin_s = lin[order]
    vv_s = vv[order]
    ii_s = ii[order]
    jj_s = jj[order]
    starts = jnp.concatenate([jnp.array([True]), lin_s[1:] != lin_s[:-1]])
    seg = jnp.cumsum(starts) - 1
    m = vv_s.shape[0]
    cvals = jax.ops.segment_sum(vv_s, seg, num_segments=m)
    rows = jax.ops.segment_max(ii_s, seg, num_segments=m)
    cols = jax.ops.segment_max(jj_s, seg, num_segments=m)
    indices = jnp.stack([rows, cols], axis=0)
    return indices, cvals


# speedup vs baseline: 26.4679x; 26.4368x over previous
import functools

import jax
import jax.numpy as jnp
from jax import lax
from jax.experimental import pallas as pl
from jax.experimental.pallas import tpu as pltpu
from jax.experimental.pallas import tpu_sc as plsc

N = 100000
E = 6400000
M = 2 * E
NW = 32
CH = M // NW
ECH = E // 16
NBINS = 102400
RNG = NBINS // NW
WS = 3200
NGS = WS // 16
NWINS = CH // WS
KS = WS // 128
WE = 3200
NGE = WE // 16
NWINE = CH // WE
BUDGET = 1000.0
EPS_POS = 1e-10
EPS_NORM = 1e-08
I32 = jnp.int32
F32 = jnp.float32
HI_SENTINEL = -2147483648

_mesh = plsc.VectorSubcoreMesh(core_axis_name="c", subcore_axis_name="s")
_cp = pltpu.CompilerParams(needs_layout_passes=False)


def _wid():
    return (lax.axis_index("s") * 2 + lax.axis_index("c")).astype(I32)


def _iota16():
    return lax.iota(I32, 16)


def _ones16():
    return jnp.full((16,), 1, I32)


def _zero_ref(ref, n, dtype):
    z = jnp.zeros((16,), dtype)

    def body(q, _):
        ref[pl.ds(q * 16, 16)] = z
        return I32(0)

    lax.fori_loop(I32(0), I32(n // 16), body, I32(0))


def _sput(ref, pos, val):
    plsc.store_scatter(ref, [jnp.full((16,), pos, I32)],
                       jnp.zeros((16,), ref.dtype) + val)


def _sget(ref, pos):
    v = plsc.load_gather(ref, [jnp.full((16,), pos, I32)])
    return jnp.max(v)


def _slice0(ref, pos, lowf):
    v = ref[pl.ds(pos, 16)]
    return jnp.max(jnp.where(_iota16() == 0, v, jnp.full((16,), lowf, ref.dtype)))


@functools.partial(
    pl.kernel,
    out_type=jax.ShapeDtypeStruct((NW, NBINS), I32),
    mesh=_mesh, compiler_params=_cp,
    scratch_types=[pltpu.VMEM((NBINS,), I32), pltpu.VMEM((WS,), I32),
                   pltpu.VMEM((WS,), I32)],
)
def _hist1(i_hbm, j_hbm, t_out, hist, buf, buf2):
    wid = _wid()
    _zero_ref(hist, NBINS, I32)
    upper = wid < 16
    srcoff = jnp.where(upper, wid, wid - 16) * I32(ECH)

    def win(w, _):
        off = (srcoff + w * I32(WS)).astype(I32)
        pltpu.sync_copy(j_hbm.at[pl.ds(off, WS)], buf)
        pltpu.sync_copy(i_hbm.at[pl.ds(off, WS)], buf2)

        def grp(q, _):
            x = jnp.where(upper, buf[pl.ds(q * 16, 16)], buf2[pl.ds(q * 16, 16)])
            plsc.addupdate_scatter(hist, [x], _ones16())
            return I32(0)

        lax.fori_loop(I32(0), I32(NGS), grp, I32(0))
        return I32(0)

    lax.fori_loop(I32(0), I32(NWINS), win, I32(0))
    pltpu.sync_copy(hist, t_out.at[wid])


@functools.partial(
    pl.kernel,
    out_type=jax.ShapeDtypeStruct((NW, NBINS), I32),
    mesh=_mesh, compiler_params=_cp,
    scratch_types=[pltpu.VMEM((NBINS,), I32), pltpu.VMEM((WS,), I32)],
)
def _hist2(r_hbm, t_out, hist, buf):
    wid = _wid()
    _zero_ref(hist, NBINS, I32)
    base = wid * I32(CH)

    def win(w, _):
        off = (base + w * I32(WS)).astype(I32)
        pltpu.sync_copy(r_hbm.at[pl.ds(off, WS)], buf)

        def grp(q, _):
            x = buf[pl.ds(q * 16, 16)]
            plsc.addupdate_scatter(hist, [x], _ones16())
            return I32(0)

        lax.fori_loop(I32(0), I32(NGS), grp, I32(0))
        return I32(0)

    lax.fori_loop(I32(0), I32(NWINS), win, I32(0))
    pltpu.sync_copy(hist, t_out.at[wid])


@functools.partial(
    pl.kernel,
    out_type=(jax.ShapeDtypeStruct((NBINS,), I32),
              jax.ShapeDtypeStruct((NW, 16), I32)),
    mesh=_mesh, compiler_params=_cp,
    scratch_types=[pltpu.VMEM((RNG,), I32), pltpu.VMEM((RNG,), I32),
                   pltpu.VMEM((16,), I32)],
)
def _scan_a(t_hbm, cs_out, rt_out, acc, row, tmp16):
    wid = _wid()
    b0 = (wid * I32(RNG)).astype(I32)
    _zero_ref(acc, RNG, I32)
    for tp in range(NW):
        pltpu.sync_copy(t_hbm.at[I32(tp), pl.ds(b0, RNG)], row)

        def add(q, _):
            acc[pl.ds(q * 16, 16)] = acc[pl.ds(q * 16, 16)] + row[pl.ds(q * 16, 16)]
            return I32(0)

        lax.fori_loop(I32(0), I32(RNG // 16), add, I32(0))

    def tot(q, racc):
        return racc + acc[pl.ds(q * 16, 16)]

    racc = lax.fori_loop(I32(0), I32(RNG // 16), tot, jnp.zeros((16,), I32))
    total = jnp.sum(racc, dtype=I32)
    pltpu.sync_copy(acc, cs_out.at[pl.ds(b0, RNG)])
    tmp16[...] = jnp.zeros((16,), I32) + total
    pltpu.sync_copy(tmp16, rt_out.at[wid])


@functools.partial(
    pl.kernel,
    out_type=jax.ShapeDtypeStruct((NW, NBINS), I32),
    mesh=_mesh, compiler_params=_cp,
    scratch_types=[pltpu.VMEM((NW, RNG), I32), pltpu.VMEM((RNG,), I32),
                   pltpu.VMEM((NW, 16), I32)],
)
def _scan_b(t_hbm, cs_hbm, rt_hbm, off_out, cnt, colsum, rtv):
    wid = _wid()
    b0 = (wid * I32(RNG)).astype(I32)
    for tp in range(NW):
        pltpu.sync_copy(t_hbm.at[I32(tp), pl.ds(b0, RNG)], cnt.at[I32(tp)])
    pltpu.sync_copy(cs_hbm.at[pl.ds(b0, RNG)], colsum)
    pltpu.sync_copy(rt_hbm, rtv)
    ta = plsc.load_gather(rtv, [_iota16(), jnp.zeros((16,), I32)])
    tb = plsc.load_gather(rtv, [_iota16() + 16, jnp.zeros((16,), I32)])
    base0 = (jnp.sum(jnp.where(_iota16() < wid, ta, 0), dtype=I32)
             + jnp.sum(jnp.where(_iota16() + 16 < wid, tb, 0), dtype=I32))

    def grp(q, sbase):
        cs16 = colsum[pl.ds(q * 16, 16)]
        incl = plsc.cumsum(cs16)
        excl = incl - cs16 + sbase

        rsum = jnp.zeros((16,), I32)
        for tp in range(NW):
            c16 = cnt[I32(tp), pl.ds(q * 16, 16)]
            cnt[I32(tp), pl.ds(q * 16, 16)] = excl + rsum
            rsum = rsum + c16
        return sbase + jnp.sum(cs16, dtype=I32)

    lax.fori_loop(I32(0), I32(RNG // 16), grp, base0)
    for tp in range(NW):
        pltpu.sync_copy(cnt.at[I32(tp)], off_out.at[I32(tp), pl.ds(b0, RNG)])


@functools.partial(
    pl.kernel,
    out_type=(jax.ShapeDtypeStruct((M,), I32),
              jax.ShapeDtypeStruct((M,), I32),
              jax.ShapeDtypeStruct((M,), F32),
              jax.ShapeDtypeStruct((NW, 16), F32)),
    mesh=_mesh, compiler_params=_cp,
    scratch_types=[pltpu.VMEM((NBINS,), I32),
                   pltpu.VMEM((WS,), I32), pltpu.VMEM((WS,), I32),
                   pltpu.VMEM((WS,), F32), pltpu.VMEM((WS,), F32),
                   pltpu.VMEM((WS,), I32), pltpu.VMEM((WS,), I32),
                   pltpu.VMEM((KS, 128), I32), pltpu.VMEM((16,), F32),
                   pltpu.SemaphoreType.DMA],
)
def _scat1(i_hbm, j_hbm, d_hbm, cost_hbm, off_hbm,
           r_out, c_out, v_out, sp_out,
           olocal, iwin, jwin, dwin, cwin, sr, sc, idxb, accf, sem):
    wid = _wid()
    pltpu.sync_copy(off_hbm.at[wid], olocal)
    upper = wid < 16
    srcoff = jnp.where(upper, wid, wid - 16) * I32(ECH)
    accf[...] = jnp.zeros((16,), F32)
    selp = jnp.where(upper, jnp.full((16,), 1.0, F32), jnp.zeros((16,), F32))

    def win(w, _):
        off = (srcoff + w * I32(WS)).astype(I32)
        pltpu.sync_copy(i_hbm.at[pl.ds(off, WS)], iwin)
        pltpu.sync_copy(j_hbm.at[pl.ds(off, WS)], jwin)
        pltpu.sync_copy(d_hbm.at[pl.ds(off, WS)], dwin)
        pltpu.sync_copy(cost_hbm.at[pl.ds(off, WS)], cwin)

        def grp(q, _):
            iv = iwin[pl.ds(q * 16, 16)]
            jv = jwin[pl.ds(q * 16, 16)]
            dv = dwin[pl.ds(q * 16, 16)]
            r16 = jnp.where(upper, iv, jv)
            c16 = jnp.where(upper, jv, iv)
            prior = plsc.load_gather(olocal, [c16])
            cntv, _lm = plsc.scan_count(c16)
            pos = prior + cntv - 1
            plsc.addupdate_scatter(olocal, [c16], _ones16())
            v = (dv * dv + EPS_POS) - EPS_POS
            dwin[pl.ds(q * 16, 16)] = v
            sr[pl.ds(q * 16, 16)] = r16
            sc[pl.ds(q * 16, 16)] = c16
            accf[...] = accf[...] + v * cwin[pl.ds(q * 16, 16)] * selp
            b = q // 8
            o = (q % 8) * 16
            idxb[b, pl.ds(o, 16)] = pos
            return I32(0)

        lax.fori_loop(I32(0), I32(NGS), grp, I32(0))

        ds_ = []
        for k in range(KS):
            s = pl.ds(k * 128, 128)
            ds_.append(pltpu.async_copy(sr.at[s], r_out.at[idxb.at[I32(k)]], sem))
            ds_.append(pltpu.async_copy(sc.at[s], c_out.at[idxb.at[I32(k)]], sem))
            ds_.append(pltpu.async_copy(dwin.at[s], v_out.at[idxb.at[I32(k)]], sem))
            if (k % 8 == 7) or k == KS - 1:
                for d in ds_:
                    d.wait()
                ds_ = []

        return I32(0)

    lax.fori_loop(I32(0), I32(NWINS), win, I32(0))
    pltpu.sync_copy(accf, sp_out.at[wid])


@functools.partial(
    pl.kernel,
    out_type=(jax.ShapeDtypeStruct((M,), I32),
              jax.ShapeDtypeStruct((M,), I32),
              jax.ShapeDtypeStruct((M,), F32)),
    mesh=_mesh, compiler_params=_cp,
    scratch_types=[pltpu.VMEM((NBINS,), I32),
                   pltpu.VMEM((WS,), I32), pltpu.VMEM((WS,), I32),
                   pltpu.VMEM((WS,), F32),
                   pltpu.VMEM((KS, 128), I32),
                   pltpu.SemaphoreType.DMA],
)
def _scat2(r_hbm, c_hbm, v_hbm, off_hbm,
           r_out, c_out, v_out,
           olocal, rwin, cwin, vwin, idxb, sem):
    wid = _wid()
    pltpu.sync_copy(off_hbm.at[wid], olocal)
    base = wid * I32(CH)

    def win(w, _):
        off = (base + w * I32(WS)).astype(I32)
        pltpu.sync_copy(r_hbm.at[pl.ds(off, WS)], rwin)
        pltpu.sync_copy(c_hbm.at[pl.ds(off, WS)], cwin)
        pltpu.sync_copy(v_hbm.at[pl.ds(off, WS)], vwin)

        def grp(q, _):
            r16 = rwin[pl.ds(q * 16, 16)]
            prior = plsc.load_gather(olocal, [r16])
            cntv, _lm = plsc.scan_count(r16)
            pos = prior + cntv - 1
            plsc.addupdate_scatter(olocal, [r16], _ones16())
            b = q // 8
            o = (q % 8) * 16
            idxb[b, pl.ds(o, 16)] = pos
            return I32(0)

        lax.fori_loop(I32(0), I32(NGS), grp, I32(0))

        ds_ = []
        for k in range(KS):
            s = pl.ds(k * 128, 128)
            ds_.append(pltpu.async_copy(rwin.at[s], r_out.at[idxb.at[I32(k)]], sem))
            ds_.append(pltpu.async_copy(cwin.at[s], c_out.at[idxb.at[I32(k)]], sem))
            ds_.append(pltpu.async_copy(vwin.at[s], v_out.at[idxb.at[I32(k)]], sem))
            if (k % 8 == 7) or k == KS - 1:
                for d in ds_:
                    d.wait()
                ds_ = []
        return I32(0)

    lax.fori_loop(I32(0), I32(NWINS), win, I32(0))


@functools.partial(
    pl.kernel,
    out_type=jax.ShapeDtypeStruct((NW, 16), I32),
    mesh=_mesh, compiler_params=_cp,
    scratch_types=[pltpu.VMEM((WE + 16,), I32), pltpu.VMEM((WE + 16,), I32),
                   pltpu.VMEM((16,), I32)],
)
def _count(r_hbm, c_hbm, cnt_out, rbuf, cbuf, tmp16):
    wid = _wid()
    a0 = wid * I32(CH)
    pre = pl.multiple_of(jnp.maximum(a0 - 16, 0), 16)
    pltpu.sync_copy(r_hbm.at[pl.ds(pre, 16)], rbuf.at[pl.ds(0, 16)])
    pltpu.sync_copy(c_hbm.at[pl.ds(pre, 16)], cbuf.at[pl.ds(0, 16)])
    first = wid == 0
    rbuf[pl.ds(0, 16)] = jnp.where(first, jnp.full((16,), -1, I32), rbuf[pl.ds(0, 16)])
    cbuf[pl.ds(0, 16)] = jnp.where(first, jnp.full((16,), -1, I32), cbuf[pl.ds(0, 16)])

    def win(w, acc):
        off = (a0 + w * I32(WE)).astype(I32)
        pltpu.sync_copy(r_hbm.at[pl.ds(off, WE)], rbuf.at[pl.ds(16, WE)])
        pltpu.sync_copy(c_hbm.at[pl.ds(off, WE)], cbuf.at[pl.ds(16, WE)])

        def grp(q, a):
            r16 = rbuf[pl.ds(16 + q * 16, 16)]
            rp = rbuf[pl.ds(15 + q * 16, 16)]
            c16 = cbuf[pl.ds(16 + q * 16, 16)]
            cp = cbuf[pl.ds(15 + q * 16, 16)]
            st = jnp.logical_or(r16 != rp, c16 != cp)
            return a + st.astype(I32)

        acc = lax.fori_loop(I32(0), I32(NGE), grp, acc)
        rbuf[pl.ds(0, 16)] = rbuf[pl.ds(WE, 16)]
        cbuf[pl.ds(0, 16)] = cbuf[pl.ds(WE, 16)]
        return acc

    acc = lax.fori_loop(I32(0), I32(NWINE), win, jnp.zeros((16,), I32))
    tmp16[...] = acc
    pltpu.sync_copy(tmp16, cnt_out.at[wid])


NBMAX = WE // 128 + 1


@functools.partial(
    pl.kernel,
    out_type=(jax.ShapeDtypeStruct((M + 128,), F32),
              jax.ShapeDtypeStruct((M + 128,), I32),
              jax.ShapeDtypeStruct((M + 128,), I32),
              jax.ShapeDtypeStruct((NW, 16), F32)),
    mesh=_mesh, compiler_params=_cp,
    scratch_types=[pltpu.VMEM((WE + 32,), I32), pltpu.VMEM((WE + 32,), I32),
                   pltpu.VMEM((WE,), F32),
                   pltpu.VMEM((WE + 16,), F32), pltpu.VMEM((WE + 16,), I32),
                   pltpu.VMEM((WE + 16,), I32),
                   pltpu.VMEM((WE + 128,), F32), pltpu.VMEM((WE + 128,), I32),
                   pltpu.VMEM((WE + 128,), I32),
                   pltpu.VMEM((NBMAX, 128), I32),
                   pltpu.VMEM((NW, 16), I32), pltpu.VMEM((32,), I32),
                   pltpu.VMEM((NW, 16), F32), pltpu.VMEM((16,), F32),
                   pltpu.SemaphoreType.DMA],
)
def _emit(r_hbm, c_hbm, v_hbm, cnt_hbm, sp_hbm,
          cv_out, rl_out, cl_out, part_out,
          rbuf, cbuf, vbuf, segsum, segr, segc,
          estv, estr, estc, idxb, cntv, totbuf, spv, pbuf, sem):
    wid = _wid()
    a0 = wid * I32(CH)
    pltpu.sync_copy(cnt_hbm, cntv)
    for tp in range(NW):
        row = cntv[I32(tp), pl.ds(0, 16)]
        _sput(totbuf, I32(tp), jnp.sum(row, dtype=I32))
    ta = totbuf[pl.ds(0, 16)]
    tb = totbuf[pl.ds(16, 16)]
    base_t = (jnp.sum(jnp.where(_iota16() < wid, ta, 0), dtype=I32)
              + jnp.sum(jnp.where(_iota16() + 16 < wid, tb, 0), dtype=I32))
    pltpu.sync_copy(sp_hbm, spv)
    sacc = jnp.zeros((16,), F32)
    for tp in range(16):
        sacc = sacc + spv[I32(tp), pl.ds(0, 16)]
    s = jnp.sum(sacc)
    svec = jnp.zeros((16,), F32) + s
    scale = jnp.full((16,), BUDGET, F32) / jnp.maximum(svec, EPS_NORM)

    pre = pl.multiple_of(jnp.maximum(a0 - 16, 0), 16)
    pltpu.sync_copy(r_hbm.at[pl.ds(pre, 16)], rbuf.at[pl.ds(0, 16)])
    pltpu.sync_copy(c_hbm.at[pl.ds(pre, 16)], cbuf.at[pl.ds(0, 16)])
    first = wid == 0
    rbuf[pl.ds(0, 16)] = jnp.where(first, jnp.full((16,), -1, I32), rbuf[pl.ds(0, 16)])
    cbuf[pl.ds(0, 16)] = jnp.where(first, jnp.full((16,), -1, I32), cbuf[pl.ds(0, 16)])

    _sput(segsum, I32(0), jnp.float32(0.0))
    _sput(segr, I32(0), I32(0))
    _sput(segc, I32(0), I32(0))
    pbuf[...] = jnp.zeros((16,), F32)

    def emit_batches(wp, cnt):
        nb = (cnt + 127) // 128

        def fill(m, _):
            g = wp + m * 16 + _iota16()
            g = jnp.where(m * 16 + _iota16() < cnt, g, I32(M))
            b = m // 8
            o = (m % 8) * 16
            idxb[b, pl.ds(o, 16)] = g
            return I32(0)

        lax.fori_loop(I32(0), (cnt + 15) // 16, fill, I32(0))
        def padrow(m, _):
            b = m // 8
            o = (m % 8) * 16
            idxb[b, pl.ds(o, 16)] = jnp.full((16,), M, I32)
            return I32(0)

        lax.fori_loop((cnt + 15) // 16, nb * 8, padrow, I32(0))

        def dma(k, _):
            s_ = pl.ds(k * 128, 128)
            pltpu.async_copy(estv.at[s_], cv_out.at[idxb.at[k]], sem).wait()
            pltpu.async_copy(estr.at[s_], rl_out.at[idxb.at[k]], sem).wait()
            pltpu.async_copy(estc.at[s_], cl_out.at[idxb.at[k]], sem).wait()
            return I32(0)

        lax.fori_loop(I32(0), nb, dma, I32(0))

    def win(w, carry):
        cum_starts, wp = carry
        off = (a0 + w * I32(WE)).astype(I32)
        pltpu.sync_copy(r_hbm.at[pl.ds(off, WE)], rbuf.at[pl.ds(16, WE)])
        pltpu.sync_copy(c_hbm.at[pl.ds(off, WE)], cbuf.at[pl.ds(16, WE)])
        pltpu.sync_copy(v_hbm.at[pl.ds(off, WE)], vbuf)

        def zz(q, _):
            segsum[pl.ds(16 + q * 16, 16)] = jnp.zeros((16,), F32)
            return I32(0)

        lax.fori_loop(I32(0), I32(WE // 16), zz, I32(0))
        segsum[pl.ds(1, 16)] = jnp.zeros((16,), F32)

        def grp(q, cin):
            r16 = rbuf[pl.ds(16 + q * 16, 16)]
            rp = rbuf[pl.ds(15 + q * 16, 16)]
            c16 = cbuf[pl.ds(16 + q * 16, 16)]
            cp = cbuf[pl.ds(15 + q * 16, 16)]
            rn = rbuf[pl.ds(17 + q * 16, 16)]
            cn = cbuf[pl.ds(17 + q * 16, 16)]
            v16 = vbuf[pl.ds(q * 16, 16)]
            it = _iota16()
            st = jnp.logical_or(r16 != rp, c16 != cp)
            L16 = plsc.cumsum(st.astype(I32)) + cin
            plsc.store_scatter(segr, [L16], r16)
            plsc.store_scatter(segc, [L16], c16)
            cg = plsc.cumsum(v16)
            pfx = cg - v16
            isf = jnp.logical_or(st, it == 0)
            basep = plsc.cummax(jnp.where(isf, pfx, jnp.full((16,), -1.0, F32)))
            part = cg - basep
            nxt = jnp.logical_or(rn != r16, cn != c16)
            flush = jnp.logical_or(nxt, it == 15)
            val = jnp.where(flush, part, jnp.zeros((16,), F32))
            plsc.addupdate_scatter(segsum, [L16], val)
            return jnp.max(L16)

        win_last = lax.fori_loop(I32(0), I32(NGE), grp, I32(0))
        cap = jnp.logical_and(cum_starts == 0, win_last > 0)
        pbuf[...] = jnp.where(cap,
                              jnp.zeros((16,), F32) + _slice0(segsum, I32(0), -1.0),
                              pbuf[...])

        s0 = jnp.where(cum_starts > 0, I32(0), I32(1))
        cnt = jnp.maximum(win_last - s0, 0)

        def stage(m, _):
            sl = pl.ds(s0 + m * 16, 16)
            estv[pl.ds(m * 16, 16)] = segsum[sl] * scale
            estr[pl.ds(m * 16, 16)] = segr[sl]
            estc[pl.ds(m * 16, 16)] = segc[sl]
            return I32(0)

        lax.fori_loop(I32(0), (cnt + 15) // 16, stage, I32(0))
        emit_batches(wp, cnt)
        sv = _slice0(segsum, win_last, -1.0)
        sr = _slice0(segr, win_last, -1)
        sc_ = _slice0(segc, win_last, -1)
        _sput(segsum, I32(0), sv)
        _sput(segr, I32(0), sr)
        _sput(segc, I32(0), sc_)
        rbuf[pl.ds(0, 16)] = rbuf[pl.ds(WE, 16)]
        cbuf[pl.ds(0, 16)] = cbuf[pl.ds(WE, 16)]
        return (cum_starts + win_last, wp + cnt)

    cum_starts, wp = lax.fori_loop(I32(0), I32(NWINE), win,
                                   (I32(0), base_t))

    pbuf[...] = jnp.where(cum_starts == 0,
                          jnp.zeros((16,), F32) + _slice0(segsum, I32(0), -1.0),
                          pbuf[...])
    estv[pl.ds(0, 16)] = (jnp.zeros((16,), F32) + _slice0(segsum, I32(0), -1.0)) * scale
    estr[pl.ds(0, 16)] = jnp.zeros((16,), I32) + _slice0(segr, I32(0), -1)
    estc[pl.ds(0, 16)] = jnp.zeros((16,), I32) + _slice0(segc, I32(0), -1)
    emit_batches(wp, jnp.where(cum_starts > 0, I32(1), I32(0)))

    pbuf[...] = pbuf[...] * scale
    pltpu.sync_copy(pbuf, part_out.at[wid])


@functools.partial(
    pl.kernel,
    out_type=(jax.ShapeDtypeStruct((M,), F32),
              jax.ShapeDtypeStruct((M,), I32),
              jax.ShapeDtypeStruct((M,), I32),
              jax.ShapeDtypeStruct((M,), I32)),
    mesh=_mesh, compiler_params=_cp,
    scratch_types=[pltpu.VMEM((WE,), F32), pltpu.VMEM((WE,), I32),
                   pltpu.VMEM((WE,), I32), pltpu.VMEM((WE,), I32),
                   pltpu.VMEM((NW, 16), I32), pltpu.VMEM((32,), I32),
                   pltpu.VMEM((NW, 16), F32)],
)
def _final(cvg_hbm, rlg_hbm, clg_hbm, cnt_hbm, part_hbm,
           cv_out, rl_out, cl_out, hi_out,
           cwin, rwin, cwin2, hwin, cntv, totbuf, partv):
    wid = _wid()
    a0 = wid * I32(CH)
    pltpu.sync_copy(cnt_hbm, cntv)
    for tp in range(NW):
        row = cntv[I32(tp), pl.ds(0, 16)]
        _sput(totbuf, I32(tp), jnp.sum(row, dtype=I32))
    ta = totbuf[pl.ds(0, 16)]
    tb = totbuf[pl.ds(16, 16)]
    ia = plsc.cumsum(ta)
    ib = plsc.cumsum(tb) + jnp.sum(ta, dtype=I32)
    U = jnp.sum(ta, dtype=I32) + jnp.sum(tb, dtype=I32)
    tga = ia - ta - 1
    tgb = ib - tb - 1
    pltpu.sync_copy(part_hbm, partv)
    pa = plsc.load_gather(partv, [_iota16(), jnp.zeros((16,), I32)])
    pb = plsc.load_gather(partv, [_iota16() + 16, jnp.zeros((16,), I32)])

    def win(w, _):
        a = (a0 + w * I32(WE)).astype(I32)
        pltpu.sync_copy(cvg_hbm.at[pl.ds(a, WE)], cwin)
        pltpu.sync_copy(rlg_hbm.at[pl.ds(a, WE)], rwin)
        pltpu.sync_copy(clg_hbm.at[pl.ds(a, WE)], cwin2)
        ma = jnp.logical_and(tga >= a, tga < a + WE)
        mb = jnp.logical_and(tgb >= a, tgb < a + WE)
        za = jnp.zeros((16,), F32)
        plsc.addupdate_scatter(cwin, [jnp.where(ma, tga - a, 0)],
                               jnp.where(ma, pa, za))
        plsc.addupdate_scatter(cwin, [jnp.where(mb, tgb - a, 0)],
                               jnp.where(mb, pb, za))

        def grp(q, _):
            p16 = a + q * 16 + _iota16()
            valid = p16 < U
            cwin[pl.ds(q * 16, 16)] = jnp.where(valid, cwin[pl.ds(q * 16, 16)], 0.0)
            rwin[pl.ds(q * 16, 16)] = jnp.where(valid, rwin[pl.ds(q * 16, 16)], 0)
            cwin2[pl.ds(q * 16, 16)] = jnp.where(valid, cwin2[pl.ds(q * 16, 16)], 0)
            hwin[pl.ds(q * 16, 16)] = jnp.where(valid, jnp.zeros((16,), I32), jnp.full((16,), HI_SENTINEL, I32))
            return I32(0)

        lax.fori_loop(I32(0), I32(NGE), grp, I32(0))
        pltpu.sync_copy(cwin, cv_out.at[pl.ds(a, WE)])
        pltpu.sync_copy(rwin, rl_out.at[pl.ds(a, WE)])
        pltpu.sync_copy(cwin2, cl_out.at[pl.ds(a, WE)])
        pltpu.sync_copy(hwin, hi_out.at[pl.ds(a, WE)])
        return I32(0)

    lax.fori_loop(I32(0), I32(NWINE), win, I32(0))


def kernel(delta_adj_raw, cost_p_sum, edge_index):
    i32i = edge_index[0].astype(I32)
    i32j = edge_index[1].astype(I32)
    d32 = delta_adj_raw.astype(F32)
    c32 = cost_p_sum.astype(F32)

    t1 = _hist1(i32i, i32j)
    cs1, rt1 = _scan_a(t1)
    off1 = _scan_b(t1, cs1, rt1)
    r1, c1, v1, sp = _scat1(i32i, i32j, d32, c32, off1)
    t2 = _hist2(r1)
    cs2, rt2 = _scan_a(t2)
    off2 = _scan_b(t2, cs2, rt2)
    r2, c2, v2 = _scat2(r1, c1, v1, off2)
    cnt = _count(r2, c2)
    cvg, rlg, clg, part = _emit(r2, c2, v2, cnt, sp)
    cv, rl, cl, hi = _final(cvg, rlg, clg, cnt, part)

    rows64 = lax.bitcast_convert_type(jnp.stack([rl, hi], axis=-1), jnp.int64)
    cols64 = lax.bitcast_convert_type(jnp.stack([cl, hi], axis=-1), jnp.int64)
    indices = jnp.stack([rows64, cols64], axis=0)
    return indices, cv
